# baseline (speedup 1.0000x reference)
import functools
import math

import jax
import jax.numpy as jnp
from jax import lax
from jax.experimental import pallas as pl
from jax.experimental.pallas import tpu as pltpu
from jax.experimental.pallas import tpu_sc as plsc

D_MODEL = 64
SCALE = math.sqrt(D_MODEL)
LANES = 16
CHUNK = 80
NBUF = 4


@functools.lru_cache(maxsize=None)
def _build_transpose(nw: int, nc: int, vocab: int):
    mesh = plsc.VectorSubcoreMesh(core_axis_name="c", subcore_axis_name="s")
    D = D_MODEL
    full_tiles = vocab // 128
    tail = vocab % 128
    tail_owner = full_tiles % nw
    extra = full_tiles % nw
    base_n = full_tiles // nw

    @functools.partial(
        pl.kernel,
        out_type=jax.ShapeDtypeStruct((vocab, 128), jnp.float32),
        mesh=mesh,
        scratch_types=[
            pltpu.VMEM((2, D, 128), jnp.float32),
            pltpu.VMEM((2, 128, 128), jnp.float32),
            pltpu.VMEM((max(tail, 8), D), jnp.float32),
            pltpu.VMEM((max(tail, 8), 128), jnp.float32),
        ]
        + [pltpu.SemaphoreType.DMA] * 4,
        compiler_params=pltpu.CompilerParams(needs_layout_passes=False),
    )
    def k(tt_hbm, tp_hbm, tin, tout, tin2, tout2, *sems):
        gsem, ssem = sems[:2], sems[2:]
        wid = lax.axis_index("s") * nc + lax.axis_index("c")
        iota = lax.iota(jnp.int32, 16)
        rowv = [iota + f0 * 16 for f0 in range(D // 16)]
        zero16 = jnp.zeros((16,), jnp.int32)
        n = lax.select(wid < extra, base_n + 1, base_n)

        def load_tile(i, b):
            t = wid + i * nw
            pltpu.async_copy(
                tt_hbm.at[:, pl.ds(t * 128, 128)],
                tin.at[b],
                gsem[b],
            )

        @pl.when(n > 0)
        def _():
            load_tile(0, 0)

        @pl.loop(0, n)
        def _tile(i):
            for bb in range(2):
                @pl.when(lax.rem(i, 2) == bb)
                def _():
                    t = wid + i * nw

                    @pl.when(i + 1 < n)
                    def _():
                        load_tile(i + 1, 1 - bb)

                    pltpu.make_async_copy(
                        tt_hbm.at[:, pl.ds(t * 128, 128)],
                        tin.at[bb],
                        gsem[bb],
                    ).wait()

                    @pl.when(i >= 2)
                    def _():
                        pltpu.make_async_copy(
                            tout.at[bb],
                            tp_hbm.at[pl.ds(t * 128, 128)],
                            ssem[bb],
                        ).wait()

                    @pl.loop(0, 128, unroll=8)
                    def _row(l):
                        lv = zero16 + l
                        for f0 in range(D // 16):
                            v = plsc.load_gather(tin.at[bb], [rowv[f0], lv])
                            tout[bb, l, pl.ds(f0 * 16, 16)] = v

                    pltpu.async_copy(
                        tout.at[bb],
                        tp_hbm.at[pl.ds(t * 128, 128)],
                        ssem[bb],
                    )

        for bb in range(2):
            @pl.when(n > bb)
            def _():
                pltpu.make_async_copy(
                    tout.at[bb], tp_hbm.at[pl.ds(0, 128)], ssem[bb]
                ).wait()

        if tail:
            @pl.when(wid == tail_owner)
            def _():
                base = full_tiles * 128
                pltpu.sync_copy(tt_hbm.at[:, pl.ds(base, tail)], tin2)

                @pl.loop(0, tail)
                def _row(l):
                    lv = zero16 + l
                    for f0 in range(D // 16):
                        v = plsc.load_gather(tin2, [rowv[f0], lv])
                        tout2[l, pl.ds(f0 * 16, 16)] = v

                pltpu.sync_copy(
                    tout2.at[pl.ds(0, tail)], tp_hbm.at[pl.ds(base, tail)]
                )

    return k


@functools.lru_cache(maxsize=None)
def _build_gather(nw: int, nc: int, nids: int):
    mesh = plsc.VectorSubcoreMesh(core_axis_name="c", subcore_axis_name="s")
    D = D_MODEL
    ids_w = nids // nw
    nchunks = ids_w // CHUNK
    nbuf = next(b for b in (NBUF, 2, 1) if nchunks % b == 0)

    @functools.partial(
        pl.kernel,
        out_type=jax.ShapeDtypeStruct((nids, D), jnp.float32),
        mesh=mesh,
        scratch_types=[
            pltpu.VMEM((ids_w,), jnp.int32),
            pltpu.VMEM((nbuf, CHUNK, 128), jnp.float32),
            pltpu.VMEM((nbuf, CHUNK, D), jnp.float32),
        ]
        + [pltpu.SemaphoreType.DMA] * (2 * nbuf),
    )
    def k(idx_hbm, table_hbm, out_hbm, idx_v, gbuf, obuf, *sems):
        gsem, ssem = sems[:nbuf], sems[nbuf:]
        wid = lax.axis_index("s") * nc + lax.axis_index("c")
        base = wid * ids_w
        pltpu.sync_copy(idx_hbm.at[pl.ds(base, ids_w)], idx_v)

        def gather(c, b):
            pltpu.async_copy(
                table_hbm.at[idx_v.at[pl.ds(c * CHUNK, CHUNK)]],
                gbuf.at[b],
                gsem[b],
            )

        for b in range(nbuf):
            gather(b, b)

        @pl.loop(0, nchunks, step=nbuf)
        def _outer(c0):
            for b in range(nbuf):
                c = c0 + b
                pltpu.make_async_copy(
                    table_hbm.at[idx_v.at[pl.ds(c * CHUNK, CHUNK)]],
                    gbuf.at[b],
                    gsem[b],
                ).wait()

                @pl.when(c0 > 0)
                def _():
                    pltpu.make_async_copy(
                        obuf.at[b],
                        out_hbm.at[pl.ds(base + c * CHUNK, CHUNK)],
                        ssem[b],
                    ).wait()

                @pl.loop(0, CHUNK, unroll=8)
                def _row(i):
                    for j in range(D // LANES):
                        sl = pl.ds(j * LANES, LANES)
                        obuf[b, i, sl] = gbuf[b, i, sl] * SCALE

                pltpu.async_copy(
                    obuf.at[b],
                    out_hbm.at[pl.ds(base + c * CHUNK, CHUNK)],
                    ssem[b],
                )

                @pl.when(c0 + nbuf < nchunks)
                def _():
                    gather(c + nbuf, b)

        for b in range(nbuf):
            c = nchunks - nbuf + b
            pltpu.make_async_copy(
                obuf.at[b],
                out_hbm.at[pl.ds(base + c * CHUNK, CHUNK)],
                ssem[b],
            ).wait()

    return k


def kernel(x, table):
    info = plsc.get_sparse_core_info()
    nc, ns = info.num_cores, info.num_subcores
    nw = nc * ns
    orig_shape = x.shape
    b = x.size
    xf = x.reshape(-1).astype(jnp.int32)
    block = nw * CHUNK
    pad = (-b) % block
    if pad:
        xf = jnp.concatenate([xf, jnp.zeros((pad,), jnp.int32)])
    tp = _build_transpose(nw, nc, table.shape[0])(table.T)
    out = _build_gather(nw, nc, b + pad)(xf, tp)
    if pad:
        out = out[:b]
    return out.reshape(*orig_shape, D_MODEL)


# speedup vs baseline: 1.7801x; 1.7801x over previous
import functools
import math

import jax
import jax.numpy as jnp
from jax import lax
from jax.experimental import pallas as pl
from jax.experimental.pallas import tpu as pltpu
from jax.experimental.pallas import tpu_sc as plsc

D_MODEL = 64
SCALE = math.sqrt(D_MODEL)
LANES = 16
CHUNK = 80
NBUF = 4


@functools.lru_cache(maxsize=None)
def _build_transpose(nw: int, nc: int, vocab: int):
    mesh = plsc.VectorSubcoreMesh(core_axis_name="c", subcore_axis_name="s")
    D = D_MODEL
    full_tiles = vocab // 128
    tail = vocab % 128
    tail_owner = full_tiles % nw
    extra = full_tiles % nw
    base_n = full_tiles // nw

    @functools.partial(
        pl.kernel,
        out_type=jax.ShapeDtypeStruct((vocab, 128), jnp.float32),
        mesh=mesh,
        scratch_types=[
            pltpu.VMEM((2, D, 128), jnp.float32),
            pltpu.VMEM((2, 128, 128), jnp.float32),
            pltpu.VMEM((max(tail, 8), D), jnp.float32),
            pltpu.VMEM((max(tail, 8), 128), jnp.float32),
        ]
        + [pltpu.SemaphoreType.DMA] * 4,
        compiler_params=pltpu.CompilerParams(needs_layout_passes=False),
    )
    def k(tt_hbm, tp_hbm, tin, tout, tin2, tout2, *sems):
        gsem, ssem = sems[:2], sems[2:]
        wid = lax.axis_index("s") * nc + lax.axis_index("c")
        iota = lax.iota(jnp.int32, 16)
        rowv = [iota + f0 * 16 for f0 in range(D // 16)]
        rot = [lax.rem(iota + d, 16) for d in range(16)]
        zero16 = jnp.zeros((16,), jnp.int32)
        n = lax.select(wid < extra, base_n + 1, base_n)

        def load_tile(i, b):
            t = wid + i * nw
            pltpu.async_copy(
                tt_hbm.at[:, pl.ds(t * 128, 128)],
                tin.at[b],
                gsem[b],
            )

        @pl.when(n > 0)
        def _():
            load_tile(0, 0)

        @pl.loop(0, n)
        def _tile(i):
            for bb in range(2):
                @pl.when(lax.rem(i, 2) == bb)
                def _():
                    t = wid + i * nw

                    @pl.when(i + 1 < n)
                    def _():
                        load_tile(i + 1, 1 - bb)

                    pltpu.make_async_copy(
                        tt_hbm.at[:, pl.ds(t * 128, 128)],
                        tin.at[bb],
                        gsem[bb],
                    ).wait()

                    @pl.when(i >= 2)
                    def _():
                        pltpu.make_async_copy(
                            tout.at[bb],
                            tp_hbm.at[pl.ds(t * 128, 128)],
                            ssem[bb],
                        ).wait()

                    @pl.loop(0, 128, step=16)
                    def _blk(l0):
                        for f0 in range(D // 16):
                            for d in range(16):
                                cv = rot[d] + l0
                                v = plsc.load_gather(
                                    tin.at[bb], [rowv[f0], cv]
                                )
                                plsc.store_scatter(
                                    tout.at[bb], [cv, rowv[f0]], v
                                )

                    pltpu.async_copy(
                        tout.at[bb],
                        tp_hbm.at[pl.ds(t * 128, 128)],
                        ssem[bb],
                    )

        for bb in range(2):
            @pl.when(n > bb)
            def _():
                pltpu.make_async_copy(
                    tout.at[bb], tp_hbm.at[pl.ds(0, 128)], ssem[bb]
                ).wait()

        if tail:
            @pl.when(wid == tail_owner)
            def _():
                base = full_tiles * 128
                pltpu.sync_copy(tt_hbm.at[:, pl.ds(base, tail)], tin2)

                @pl.loop(0, tail)
                def _row(l):
                    lv = zero16 + l
                    for f0 in range(D // 16):
                        v = plsc.load_gather(tin2, [rowv[f0], lv])
                        tout2[l, pl.ds(f0 * 16, 16)] = v

                pltpu.sync_copy(
                    tout2.at[pl.ds(0, tail)], tp_hbm.at[pl.ds(base, tail)]
                )

    return k


@functools.lru_cache(maxsize=None)
def _build_gather(nw: int, nc: int, nids: int):
    mesh = plsc.VectorSubcoreMesh(core_axis_name="c", subcore_axis_name="s")
    D = D_MODEL
    ids_w = nids // nw
    nchunks = ids_w // CHUNK
    nbuf = next(b for b in (NBUF, 2, 1) if nchunks % b == 0)

    @functools.partial(
        pl.kernel,
        out_type=jax.ShapeDtypeStruct((nids, D), jnp.float32),
        mesh=mesh,
        scratch_types=[
            pltpu.VMEM((ids_w,), jnp.int32),
            pltpu.VMEM((nbuf, CHUNK, 128), jnp.float32),
            pltpu.VMEM((nbuf, CHUNK, D), jnp.float32),
        ]
        + [pltpu.SemaphoreType.DMA] * (2 * nbuf),
    )
    def k(idx_hbm, table_hbm, out_hbm, idx_v, gbuf, obuf, *sems):
        gsem, ssem = sems[:nbuf], sems[nbuf:]
        wid = lax.axis_index("s") * nc + lax.axis_index("c")
        base = wid * ids_w
        pltpu.sync_copy(idx_hbm.at[pl.ds(base, ids_w)], idx_v)

        def gather(c, b):
            pltpu.async_copy(
                table_hbm.at[idx_v.at[pl.ds(c * CHUNK, CHUNK)]],
                gbuf.at[b],
                gsem[b],
            )

        for b in range(nbuf):
            gather(b, b)

        @pl.loop(0, nchunks, step=nbuf)
        def _outer(c0):
            for b in range(nbuf):
                c = c0 + b
                pltpu.make_async_copy(
                    table_hbm.at[idx_v.at[pl.ds(c * CHUNK, CHUNK)]],
                    gbuf.at[b],
                    gsem[b],
                ).wait()

                @pl.when(c0 > 0)
                def _():
                    pltpu.make_async_copy(
                        obuf.at[b],
                        out_hbm.at[pl.ds(base + c * CHUNK, CHUNK)],
                        ssem[b],
                    ).wait()

                @pl.loop(0, CHUNK, unroll=8)
                def _row(i):
                    for j in range(D // LANES):
                        sl = pl.ds(j * LANES, LANES)
                        obuf[b, i, sl] = gbuf[b, i, sl] * SCALE

                pltpu.async_copy(
                    obuf.at[b],
                    out_hbm.at[pl.ds(base + c * CHUNK, CHUNK)],
                    ssem[b],
                )

                @pl.when(c0 + nbuf < nchunks)
                def _():
                    gather(c + nbuf, b)

        for b in range(nbuf):
            c = nchunks - nbuf + b
            pltpu.make_async_copy(
                obuf.at[b],
                out_hbm.at[pl.ds(base + c * CHUNK, CHUNK)],
                ssem[b],
            ).wait()

    return k


def kernel(x, table):
    info = plsc.get_sparse_core_info()
    nc, ns = info.num_cores, info.num_subcores
    nw = nc * ns
    orig_shape = x.shape
    b = x.size
    xf = x.reshape(-1).astype(jnp.int32)
    block = nw * CHUNK
    pad = (-b) % block
    if pad:
        xf = jnp.concatenate([xf, jnp.zeros((pad,), jnp.int32)])
    tp = _build_transpose(nw, nc, table.shape[0])(table.T)
    out = _build_gather(nw, nc, b + pad)(xf, tp)
    if pad:
        out = out[:b]
    return out.reshape(*orig_shape, D_MODEL)
